# baseline (speedup 1.0000x reference)
import functools

import jax
import jax.numpy as jnp
from jax import lax
from jax.experimental import pallas as pl
from jax.experimental.pallas import tpu as pltpu
from jax.experimental.pallas import tpu_sc as plsc

_N_BINS = 10
_ROWS = 16384
_LANES = 16
_NTILES = 32
_NCELLS = 32
_CHUNK = 16384
_UNROLL = 8
_PHASES = 2


def _row_stats(bounds_ref, x, lab):
    rows, ncls = x.shape
    xt = x.T
    m = jnp.max(xt, axis=0)
    s = jnp.sum(jnp.exp(xt - m[None, :]), axis=0)
    conf = 1.0 / s
    row = lax.broadcasted_iota(jnp.int32, (ncls, rows), 0)
    pred = jnp.min(jnp.where(xt == m[None, :], row, ncls), axis=0)
    acci = (pred == lab).astype(jnp.int32)
    binv = jnp.zeros((rows,), jnp.int32)
    for j in range(1, _N_BINS):
        binv = binv + (conf > bounds_ref[j]).astype(jnp.int32)
    return conf, binv * 2 + acci


def _dense_body(bounds_ref, lg_a, lg_b, lab_a, lab_b, conf_ref, idx_ref):
    conf, idx = _row_stats(bounds_ref, lg_a[...], lab_a[0, 0, :])
    conf_ref[0, 0, 0, :] = conf
    idx_ref[0, 0, 0, :] = idx
    conf, idx = _row_stats(bounds_ref, lg_b[...], lab_b[0, 0, :])
    conf_ref[0, 1, 0, :] = conf
    idx_ref[0, 1, 0, :] = idx


def _dense_stage(bounds, logits, labels3, blk_off, nb):
    ncls = logits.shape[1]
    nb2 = nb // 2
    off_b = blk_off + nb2
    out4 = pl.BlockSpec((1, 2, 1, _ROWS), lambda i: (i, 0, 0, 0))
    conf4, idx4 = pl.pallas_call(
        _dense_body,
        grid=(nb2,),
        in_specs=[
            pl.BlockSpec(memory_space=pltpu.SMEM),
            pl.BlockSpec((_ROWS, ncls), lambda i: (i + blk_off, 0)),
            pl.BlockSpec((_ROWS, ncls), lambda i: (i + off_b, 0)),
            pl.BlockSpec((1, 1, _ROWS), lambda i: (i + blk_off, 0, 0)),
            pl.BlockSpec((1, 1, _ROWS), lambda i: (i + off_b, 0, 0)),
        ],
        out_specs=[out4, out4],
        out_shape=[
            jax.ShapeDtypeStruct((nb2, 2, 1, _ROWS), jnp.float32),
            jax.ShapeDtypeStruct((nb2, 2, 1, _ROWS), jnp.int32),
        ],
    )(bounds, logits, logits, labels3, labels3)
    return conf4, idx4


def _make_hist_kernel(n):
    per_tile = n // _NTILES
    chunk = min(_CHUNK, per_tile)
    n_chunks = per_tile // chunk
    nacc = _LANES * _NCELLS
    mesh = plsc.VectorSubcoreMesh(core_axis_name="c", subcore_axis_name="s")

    @functools.partial(
        pl.kernel,
        out_type=jax.ShapeDtypeStruct((_NTILES, 4, _LANES), jnp.float32),
        mesh=mesh,
        compiler_params=pltpu.CompilerParams(needs_layout_passes=False),
        scratch_types=[
            pltpu.VMEM((chunk,), jnp.int32),
            pltpu.VMEM((chunk,), jnp.float32),
            pltpu.VMEM((nacc,), jnp.float32),
            pltpu.VMEM((nacc,), jnp.float32),
            pltpu.VMEM((4, _LANES), jnp.float32),
        ],
    )
    def hist(idx_hbm, conf_hbm, out_hbm,
             idx_v, conf_v, cnt_a, csum_a, res_v):
        cid = lax.axis_index("c")
        sid = lax.axis_index("s")
        wid = sid * 2 + cid
        zero16 = jnp.zeros((_LANES,), jnp.float32)
        for r in range(nacc // _LANES):
            sl = pl.ds(r * _LANES, _LANES)
            cnt_a[sl] = zero16
            csum_a[sl] = zero16
        lanes = lax.iota(jnp.int32, _LANES)
        lane_off = lanes * _NCELLS
        ones = jnp.ones((_LANES,), jnp.float32)
        base0 = wid * per_tile
        for c in range(n_chunks):
            base = base0 + c * chunk
            pltpu.sync_copy(idx_hbm.at[pl.ds(base, chunk)], idx_v)
            pltpu.sync_copy(conf_hbm.at[pl.ds(base, chunk)], conf_v)

            def body(i, _):
                for u in range(_UNROLL):
                    o = (i * _UNROLL + u) * _LANES
                    b = idx_v[pl.ds(o, _LANES)]
                    v = conf_v[pl.ds(o, _LANES)]
                    addr = lane_off + b
                    plsc.addupdate_scatter(cnt_a, [addr], ones)
                    plsc.addupdate_scatter(csum_a, [addr], v)
                return 0

            lax.fori_loop(0, chunk // (_LANES * _UNROLL), body, 0)
        tot = [zero16, zero16, zero16, zero16]
        for r in range(_LANES):
            for h in range(2):
                sl = pl.ds(r * _NCELLS + h * _LANES, _LANES)
                tot[h] = tot[h] + cnt_a[sl]
                tot[2 + h] = tot[2 + h] + csum_a[sl]
        for k in range(4):
            res_v[k] = tot[k]
        pltpu.sync_copy(res_v, out_hbm.at[wid])

    return hist


def kernel(logits, labels):
    n, _ = logits.shape
    bounds = jnp.linspace(0.0, 1.0, _N_BINS + 1).astype(jnp.float32)
    nh = n // _PHASES
    nbh = nh // _ROWS
    labels3 = labels.reshape(n // _ROWS, 1, _ROWS)
    hist = _make_hist_kernel(nh)
    parts = []
    for p in range(_PHASES):
        conf3, idx3 = _dense_stage(bounds, logits, labels3, p * nbh, nbh)
        parts.append(hist(idx3.reshape(nh), conf3.reshape(nh)))
    partials = jnp.concatenate(parts, axis=0)
    stats = jnp.sum(partials, axis=0)
    cnt_c = jnp.concatenate([stats[0], stats[1]]).reshape(_LANES, 2)
    csum_c = jnp.concatenate([stats[2], stats[3]]).reshape(_LANES, 2)
    cnt = cnt_c[:_N_BINS, 0] + cnt_c[:_N_BINS, 1]
    asum = cnt_c[:_N_BINS, 1]
    csum = csum_c[:_N_BINS, 0] + csum_c[:_N_BINS, 1]
    nf = jnp.float32(n)
    safe = jnp.maximum(cnt, 1.0)
    contrib = jnp.abs(csum / safe - asum / safe) * (cnt / nf)
    ece = jnp.sum(jnp.where(cnt > 0, contrib, 0.0), keepdims=True)
    acc = jnp.sum(cnt_c[:, 1]) / nf
    return ece, acc


# speedup vs baseline: 1.0164x; 1.0164x over previous
import functools

import jax
import jax.numpy as jnp
from jax import lax
from jax.experimental import pallas as pl
from jax.experimental.pallas import tpu as pltpu
from jax.experimental.pallas import tpu_sc as plsc

_N_BINS = 10
_ROWS = 32768
_LANES = 16
_NTILES = 32
_NCELLS = 32
_CHUNK = 16384
_UNROLL = 8
_SPLIT = 24


def _dense_body(bounds_ref, logits_ref, labels_ref, val_ref):
    x = logits_ref[...]
    rows, ncls = x.shape
    xt = x.T
    m = jnp.max(xt, axis=0)
    s = jnp.sum(jnp.exp(xt - m[None, :]), axis=0)
    conf = 1.0 / s
    row = lax.broadcasted_iota(jnp.int32, (ncls, rows), 0)
    pred = jnp.min(jnp.where(xt == m[None, :], row, ncls), axis=0)
    lab = labels_ref[0, 0, :]
    acci = (pred == lab).astype(jnp.int32)
    binv = jnp.zeros((rows,), jnp.int32)
    for j in range(1, _N_BINS):
        binv = binv + (conf > bounds_ref[j]).astype(jnp.int32)
    idx2 = binv * 2 + acci
    val_ref[0, 0, :] = idx2.astype(jnp.float32) + conf * 0.5


def _dense_stage(bounds, logits, labels3, blk_off, nb):
    ncls = logits.shape[1]
    return pl.pallas_call(
        _dense_body,
        grid=(nb,),
        in_specs=[
            pl.BlockSpec(memory_space=pltpu.SMEM),
            pl.BlockSpec((_ROWS, ncls), lambda i: (i + blk_off, 0)),
            pl.BlockSpec((1, 1, _ROWS), lambda i: (i + blk_off, 0, 0)),
        ],
        out_specs=pl.BlockSpec((1, 1, _ROWS), lambda i: (i, 0, 0)),
        out_shape=jax.ShapeDtypeStruct((nb, 1, _ROWS), jnp.float32),
    )(bounds, logits, labels3)


def _make_hist_kernel(n):
    per_tile = n // _NTILES
    n_chunks = -(-per_tile // _CHUNK)
    chunk = per_tile // n_chunks
    assert chunk * n_chunks == per_tile
    assert chunk % (_LANES * _UNROLL) == 0
    nacc = _LANES * _NCELLS
    mesh = plsc.VectorSubcoreMesh(core_axis_name="c", subcore_axis_name="s")

    @functools.partial(
        pl.kernel,
        out_type=jax.ShapeDtypeStruct((_NTILES, 4, _LANES), jnp.float32),
        mesh=mesh,
        compiler_params=pltpu.CompilerParams(needs_layout_passes=False),
        scratch_types=[
            pltpu.VMEM((chunk,), jnp.float32),
            pltpu.VMEM((nacc,), jnp.float32),
            pltpu.VMEM((nacc,), jnp.float32),
            pltpu.VMEM((4, _LANES), jnp.float32),
        ],
    )
    def hist(val_hbm, out_hbm, val_v, cnt_a, csum_a, res_v):
        cid = lax.axis_index("c")
        sid = lax.axis_index("s")
        wid = sid * 2 + cid
        zero16 = jnp.zeros((_LANES,), jnp.float32)
        for r in range(nacc // _LANES):
            sl = pl.ds(r * _LANES, _LANES)
            cnt_a[sl] = zero16
            csum_a[sl] = zero16
        lanes = lax.iota(jnp.int32, _LANES)
        lane_off = lanes * _NCELLS
        ones = jnp.ones((_LANES,), jnp.float32)
        base0 = wid * per_tile
        for c in range(n_chunks):
            pltpu.sync_copy(val_hbm.at[pl.ds(base0 + c * chunk, chunk)], val_v)

            def body(i, _):
                for u in range(_UNROLL):
                    o = (i * _UNROLL + u) * _LANES
                    v = val_v[pl.ds(o, _LANES)]
                    cell = v.astype(jnp.int32)
                    conf = (v - cell.astype(jnp.float32)) * 2.0
                    addr = lane_off + cell
                    plsc.addupdate_scatter(cnt_a, [addr], ones)
                    plsc.addupdate_scatter(csum_a, [addr], conf)
                return 0

            lax.fori_loop(0, chunk // (_LANES * _UNROLL), body, 0)
        tot = [zero16, zero16, zero16, zero16]
        for r in range(_LANES):
            for h in range(2):
                sl = pl.ds(r * _NCELLS + h * _LANES, _LANES)
                tot[h] = tot[h] + cnt_a[sl]
                tot[2 + h] = tot[2 + h] + csum_a[sl]
        for k in range(4):
            res_v[k] = tot[k]
        pltpu.sync_copy(res_v, out_hbm.at[wid])

    return hist


def kernel(logits, labels):
    n, _ = logits.shape
    bounds = jnp.linspace(0.0, 1.0, _N_BINS + 1).astype(jnp.float32)
    nb_all = n // _ROWS
    labels3 = labels.reshape(nb_all, 1, _ROWS)
    parts = []
    for blk_off, nbp in ((0, _SPLIT), (_SPLIT, nb_all - _SPLIT)):
        nh = nbp * _ROWS
        val3 = _dense_stage(bounds, logits, labels3, blk_off, nbp)
        parts.append(_make_hist_kernel(nh)(val3.reshape(nh)))
    partials = jnp.concatenate(parts, axis=0)
    stats = jnp.sum(partials, axis=0)
    cnt_c = jnp.concatenate([stats[0], stats[1]]).reshape(_LANES, 2)
    csum_c = jnp.concatenate([stats[2], stats[3]]).reshape(_LANES, 2)
    cnt = cnt_c[:_N_BINS, 0] + cnt_c[:_N_BINS, 1]
    asum = cnt_c[:_N_BINS, 1]
    csum = csum_c[:_N_BINS, 0] + csum_c[:_N_BINS, 1]
    nf = jnp.float32(n)
    safe = jnp.maximum(cnt, 1.0)
    contrib = jnp.abs(csum / safe - asum / safe) * (cnt / nf)
    ece = jnp.sum(jnp.where(cnt > 0, contrib, 0.0), keepdims=True)
    acc = jnp.sum(cnt_c[:, 1]) / nf
    return ece, acc
